# baseline (speedup 1.0000x reference)
import jax
import jax.numpy as jnp
from jax import lax
from jax.experimental import pallas as pl
from jax.experimental.pallas import tpu as pltpu
from jax.experimental.pallas import tpu_sc as plsc

_R = 6144
_C0 = 128
_W = 172
_WP = 256
_NV = 11
_N = _R * _W
_K = float(int(0.1 * _N))
_ROWS = _R // 16
_NB1 = 4096
_NB3 = 256
_MC = 256


def _sc_body(x_hbm, out_hbm, data, hist, msl, merged, pub, st, tmp,
             itmp16, sh_hists, sh_tot, sh_res):
    c = lax.axis_index("c")
    s = lax.axis_index("s")
    zeros16 = jnp.zeros((16,), jnp.float32)
    ones16 = jnp.ones((16,), jnp.float32)
    lane = lax.iota(jnp.int32, 16)
    tail_ok = lane < 12

    pltpu.sync_copy(x_hbm.at[pl.ds(s * _ROWS, _ROWS), pl.ds(_C0, _WP)], data)

    @plsc.parallel_loop(0, _ROWS, step=1, unroll=2)
    def _enc(r):
        for cc in range(_NV):
            raw = data[r, pl.ds(cc * 16, 16)]
            data[r, pl.ds(cc * 16, 16)] = raw ^ ((raw >> 31) & 0x7FFFFFFF)

    def zero_hist(n):
        @plsc.parallel_loop(0, n // 16, step=1, unroll=8)
        def _zb(i):
            hist[pl.ds(i * 16, 16)] = zeros16

    def sweep(shift, mask_c, add_c, pref_shift, pref_vec, use_pred):
        @plsc.parallel_loop(0, _ROWS, step=1, unroll=2)
        def _row(r):
            for cc in range(_NV):
                m = data[r, pl.ds(cc * 16, 16)]
                b = ((m >> shift) & mask_c) + add_c
                ok = tail_ok if cc == _NV - 1 else None
                if use_pred:
                    p = (m >> pref_shift) == pref_vec
                    ok = p if ok is None else (ok & p)
                plsc.addupdate_scatter(hist, [b], ones16, mask=ok)

    def merge_and_scan(kv, nb):
        sl = nb // 16
        pltpu.sync_copy(hist.at[pl.ds(0, nb)], sh_hists.at[s, pl.ds(0, nb)])
        plsc.subcore_barrier()
        pltpu.sync_copy(sh_hists.at[:, pl.ds(s * sl, sl)],
                        msl.at[:, pl.ds(0, sl)])

        @plsc.parallel_loop(0, sl // 16, step=1, unroll=2)
        def _mg(j):
            acc = zeros16
            for row in range(16):
                acc = acc + msl[row, pl.ds(j * 16, 16)]
            merged[pl.ds(j * 16, 16)] = acc

        def ts(j, tot):
            return tot + jnp.sum(merged[pl.ds(j * 16, 16)])
        my_tot = lax.fori_loop(0, sl // 16, ts, jnp.float32(0.0))

        for jj in range(8):
            tmp[pl.ds(jj * 16, 16)] = jnp.broadcast_to(my_tot, (16,))
        pltpu.sync_copy(tmp, sh_tot.at[s])
        plsc.subcore_barrier()
        pltpu.sync_copy(sh_tot, st)
        totals = zeros16
        for j in range(16):
            totals = jnp.where(lane == j, st[j, pl.ds(0, 16)], totals)
        base_v = plsc.cumsum(totals) - totals
        base = jnp.sum(jnp.where(lane == s, base_v, zeros16))
        mine = jnp.sum(jnp.where(lane == s, totals, zeros16))
        ks = jnp.max(kv)
        has = (base <= ks) & (ks < base + mine)

        def sc(j, carry):
            run, hcnt, below = carry
            v = merged[pl.ds(j * 16, 16)]
            cumg = plsc.cumsum(v) + (run + base)
            msk = cumg <= kv
            pc = plsc.all_reduce_population_count(msk).astype(jnp.float32)
            hcnt = hcnt + pc
            below = below + jnp.sum(jnp.where(msk, v, zeros16))
            return run + jnp.sum(v), hcnt, below
        _, hcnt, below = lax.fori_loop(
            0, sl // 16, sc, (jnp.float32(0.0), zeros16, zeros16))
        h_glob = hcnt + (s * sl).astype(jnp.float32)
        below_glob = below + base

        pub[0, pl.ds(0, 16)] = h_glob
        pub[0, pl.ds(16, 16)] = h_glob
        pub[1, pl.ds(0, 16)] = below_glob
        pub[1, pl.ds(16, 16)] = below_glob

        @pl.when(has)
        def _():
            pltpu.sync_copy(pub, sh_res)

        plsc.subcore_barrier()
        pltpu.sync_copy(sh_res, pub)
        h = pub[0, pl.ds(0, 16)].astype(jnp.int32)
        below_r = pub[1, pl.ds(0, 16)]
        return h, below_r

    kv = jnp.broadcast_to(jnp.float32(_K), (16,))

    zero_hist(_NB1)
    sweep(20, -1, 2048, 0, None, False)
    h1, bel1 = merge_and_scan(kv, _NB1)
    k2 = kv - bel1
    h1s = h1 - 2048

    zero_hist(_NB1)
    sweep(8, 0xFFF, 0, 20, h1s, True)
    h2, bel2 = merge_and_scan(k2, _NB1)
    k3 = k2 - bel2
    p2 = (h1s << 12) | h2

    zero_hist(_NB3)
    sweep(0, 0xFF, 0, 8, p2, True)
    pltpu.sync_copy(hist.at[pl.ds(0, _NB3)], sh_hists.at[s, pl.ds(0, _NB3)])
    plsc.subcore_barrier()
    pltpu.sync_copy(sh_hists.at[:, pl.ds(0, _NB3)], msl)
    run3 = jnp.float32(0.0)
    h3f = zeros16
    for j in range(_NB3 // 16):
        acc = zeros16
        for row in range(16):
            acc = acc + msl[row, pl.ds(j * 16, 16)]
        cum = plsc.cumsum(acc) + run3
        pc = plsc.all_reduce_population_count(cum <= k3)
        h3f = h3f + pc.astype(jnp.float32)
        run3 = run3 + jnp.sum(acc)
    h3 = h3f.astype(jnp.int32)

    m_eps = (p2 << 8) | h3
    b_eps = m_eps ^ ((m_eps >> 31) & 0x7FFFFFFF)
    itmp16[...] = b_eps

    @pl.when((c == 0) & (s == 0))
    def _():
        pltpu.sync_copy(itmp16, out_hbm)


def _sc_select(x2i):
    mesh = plsc.VectorSubcoreMesh(core_axis_name="c", subcore_axis_name="s",
                                  num_cores=1)
    f = pl.kernel(
        _sc_body,
        out_type=jax.ShapeDtypeStruct((16,), jnp.int32),
        mesh=mesh,
        compiler_params=pltpu.CompilerParams(needs_layout_passes=False),
        scratch_types=[
            pltpu.VMEM((_ROWS, _WP), jnp.int32),
            pltpu.VMEM((_NB1,), jnp.float32),
            pltpu.VMEM((16, _MC), jnp.float32),
            pltpu.VMEM((_NB1 // 16,), jnp.float32),
            pltpu.VMEM((2, 128), jnp.float32),
            pltpu.VMEM((16, 128), jnp.float32),
            pltpu.VMEM((128,), jnp.float32),
            pltpu.VMEM((16,), jnp.int32),
            pltpu.VMEM_SHARED((16, _NB1), jnp.float32),
            pltpu.VMEM_SHARED((16, 128), jnp.float32),
            pltpu.VMEM_SHARED((2, 128), jnp.float32),
        ],
    )
    return f(x2i)


def _tc_body(eps_ref, x_ref, o_ref):
    xs = x_ref[:, _C0:_C0 + _W]
    xmax = jnp.max(xs)
    eps = eps_ref[0]
    t = jnp.maximum(eps, jnp.float32(0.001))
    lo = jnp.log10(t)
    hi = jnp.log10(jnp.maximum(xmax, t))
    inv = 1.0 / (hi - lo)
    y = jnp.log10(jnp.maximum(xs, t))
    o_ref[...] = (y - lo) * inv


def kernel(x):
    x2 = x.reshape(_R, 512)
    eps_bits = _sc_select(lax.bitcast_convert_type(x2, jnp.int32))
    eps16 = lax.bitcast_convert_type(eps_bits, jnp.float32)
    out = pl.pallas_call(
        _tc_body,
        out_shape=jax.ShapeDtypeStruct((_R, _W), jnp.float32),
        in_specs=[
            pl.BlockSpec(memory_space=pltpu.SMEM),
            pl.BlockSpec(memory_space=pltpu.VMEM),
        ],
        out_specs=pl.BlockSpec(memory_space=pltpu.VMEM),
    )(eps16, x2)
    return out.reshape(x.shape[0], x.shape[1], _W)
